# baseline (speedup 1.0000x reference)
import functools

import jax
import jax.numpy as jnp
from jax import lax
from jax.experimental import pallas as pl
from jax.experimental.pallas import tpu as pltpu
from jax.experimental.pallas import tpu_sc as plsc

MAXR = 1000
B = 16384
NC = 2
NS = 16
NW = NC * NS
BPW = B // NW
CH = 128
NCH = BPW // CH
L = 16
MAGIC = 2.0 ** 23

_mesh = plsc.VectorSubcoreMesh(core_axis_name="c", subcore_axis_name="s")


@functools.partial(
    pl.kernel,
    mesh=_mesh,
    out_type=jax.ShapeDtypeStruct((NW, NCH, CH), jnp.float32),
    scratch_types=[
        pltpu.VMEM((NCH, CH), jnp.float32),
        pltpu.VMEM((NCH, CH), jnp.int32),
        pltpu.VMEM((NCH, CH), jnp.float32),
        pltpu.SemaphoreType.DMA,
        pltpu.SemaphoreType.DMA,
        pltpu.SemaphoreType.DMA,
        pltpu.SemaphoreType.DMA,
        pltpu.SemaphoreType.DMA,
    ],
)
def _gather_kernel(x_hbm, p_hbm, out_hbm, x_v, idx_v, g_v, sg0, sg1, sg2, sg3, so):
    wid = lax.axis_index("s") * NC + lax.axis_index("c")
    base = wid * BPW
    sg = (sg0, sg1, sg2, sg3)
    xloads = [
        pltpu.async_copy(x_hbm.at[pl.ds(base + j * CH, CH)], x_v.at[j], sg[j])
        for j in range(NCH)
    ]
    lane = lax.iota(jnp.int32, L)
    gathers = []
    for j in range(NCH):
        xloads[j].wait()
        for t in range(CH // L):
            off = j * CH + t * L
            v = x_v[j, pl.ds(t * L, L)] * float(MAXR - 1)
            r = (v + MAGIC) - MAGIC
            c = r.astype(jnp.int32)
            bconst = (((base + off) >> 7) << 10) + ((base + off) & 127)
            poff = (((c >> 3) << 17) | ((c & 7) << 7)) + (lane + bconst)
            idx_v[j, pl.ds(t * L, L)] = poff
        gathers.append(pltpu.async_copy(p_hbm.at[idx_v.at[j]], g_v.at[j], sg[j]))
    outs = []
    for j in range(NCH):
        gathers[j].wait()
        outs.append(pltpu.async_copy(g_v.at[j], out_hbm.at[wid].at[j], so))
    for o in outs:
        o.wait()


def kernel(x, param):
    p_perm = (
        param.reshape(B, MAXR)
        .transpose(1, 0)
        .reshape(MAXR // 8, 8, B // 128, 128)
        .transpose(0, 2, 1, 3)
        .reshape(B * MAXR)
    )
    out = _gather_kernel(x, p_perm)
    return out.reshape(B, 1, 1)


# speedup vs baseline: 1.0039x; 1.0039x over previous
import functools

import jax
import jax.numpy as jnp
from jax import lax
from jax.experimental import pallas as pl
from jax.experimental.pallas import tpu as pltpu
from jax.experimental.pallas import tpu_sc as plsc

MAXR = 1000
B = 16384
NC = 2
NS = 16
NW = NC * NS
BPW = B // NW
CH = 128
NCH = BPW // CH
L = 16
MAGIC = 2.0 ** 23

_mesh = plsc.VectorSubcoreMesh(core_axis_name="c", subcore_axis_name="s")


@functools.partial(
    pl.kernel,
    mesh=_mesh,
    out_type=jax.ShapeDtypeStruct((NW, NCH, CH), jnp.float32),
    scratch_types=[
        pltpu.VMEM((BPW,), jnp.float32),
        pltpu.VMEM((NCH, CH), jnp.int32),
        pltpu.VMEM((NCH, CH), jnp.float32),
        pltpu.SemaphoreType.DMA,
        pltpu.SemaphoreType.DMA,
        pltpu.SemaphoreType.DMA,
        pltpu.SemaphoreType.DMA,
        pltpu.SemaphoreType.DMA,
    ],
)
def _gather_kernel(x_hbm, p_hbm, out_hbm, x_v, idx_v, g_v, sg0, sg1, sg2, sg3, so):
    wid = lax.axis_index("s") * NC + lax.axis_index("c")
    base = wid * BPW
    sg = (sg0, sg1, sg2, sg3)
    pltpu.sync_copy(x_hbm.at[pl.ds(base, BPW)], x_v)
    lane = lax.iota(jnp.int32, L)
    gathers = []
    for j in range(NCH):
        for t in range(CH // L):
            off = j * CH + t * L
            v = x_v[pl.ds(off, L)] * float(MAXR - 1)
            r = (v + MAGIC) - MAGIC
            c = r.astype(jnp.int32)
            bconst = (((base + off) >> 7) << 10) + ((base + off) & 127)
            poff = (((c >> 3) << 17) | ((c & 7) << 7)) + (lane + bconst)
            idx_v[j, pl.ds(t * L, L)] = poff
        gathers.append(pltpu.async_copy(p_hbm.at[idx_v.at[j]], g_v.at[j], sg[j]))
    outs = []
    for j in range(NCH):
        gathers[j].wait()
        outs.append(pltpu.async_copy(g_v.at[j], out_hbm.at[wid].at[j], so))
    for o in outs:
        o.wait()


def kernel(x, param):
    p_perm = (
        param.reshape(B, MAXR)
        .transpose(1, 0)
        .reshape(MAXR // 8, 8, B // 128, 128)
        .transpose(0, 2, 1, 3)
        .reshape(B * MAXR)
    )
    out = _gather_kernel(x, p_perm)
    return out.reshape(B, 1, 1)


# speedup vs baseline: 1.0381x; 1.0340x over previous
import functools

import jax
import jax.numpy as jnp
from jax import lax
from jax.experimental import pallas as pl
from jax.experimental.pallas import tpu as pltpu
from jax.experimental.pallas import tpu_sc as plsc

MAXR = 1000
B = 16384
NC = 1
NS = 16
NW = NC * NS
BPW = B // NW
CH = 128
NCH = BPW // CH
L = 16
MAGIC = 2.0 ** 23

_mesh = plsc.VectorSubcoreMesh(core_axis_name="c", subcore_axis_name="s", num_cores=1)


@functools.partial(
    pl.kernel,
    mesh=_mesh,
    out_type=jax.ShapeDtypeStruct((NW, NCH, CH), jnp.float32),
    scratch_types=[
        pltpu.VMEM((BPW,), jnp.float32),
        pltpu.VMEM((NCH, CH), jnp.int32),
        pltpu.VMEM((NCH, CH), jnp.float32),
        pltpu.SemaphoreType.DMA,
        pltpu.SemaphoreType.DMA,
        pltpu.SemaphoreType.DMA,
        pltpu.SemaphoreType.DMA,
        pltpu.SemaphoreType.DMA,
        pltpu.SemaphoreType.DMA,
        pltpu.SemaphoreType.DMA,
        pltpu.SemaphoreType.DMA,
        pltpu.SemaphoreType.DMA,
    ],
)
def _gather_kernel(x_hbm, p_hbm, out_hbm, x_v, idx_v, g_v,
                   sg0, sg1, sg2, sg3, sg4, sg5, sg6, sg7, so):
    wid = lax.axis_index("s") * NC + lax.axis_index("c")
    base = wid * BPW
    sg = (sg0, sg1, sg2, sg3, sg4, sg5, sg6, sg7)
    pltpu.sync_copy(x_hbm.at[pl.ds(base, BPW)], x_v)
    lane = lax.iota(jnp.int32, L)
    gathers = []
    for j in range(NCH):
        for t in range(CH // L):
            off = j * CH + t * L
            v = x_v[pl.ds(off, L)] * float(MAXR - 1)
            r = (v + MAGIC) - MAGIC
            c = r.astype(jnp.int32)
            bconst = (((base + off) >> 7) << 10) + ((base + off) & 127)
            poff = (((c >> 3) << 17) | ((c & 7) << 7)) + (lane + bconst)
            idx_v[j, pl.ds(t * L, L)] = poff
        gathers.append(pltpu.async_copy(p_hbm.at[idx_v.at[j]], g_v.at[j], sg[j]))
    outs = []
    for j in range(NCH):
        gathers[j].wait()
        outs.append(pltpu.async_copy(g_v.at[j], out_hbm.at[wid].at[j], so))
    for o in outs:
        o.wait()


def kernel(x, param):
    p_perm = (
        param.reshape(B, MAXR)
        .transpose(1, 0)
        .reshape(MAXR // 8, 8, B // 128, 128)
        .transpose(0, 2, 1, 3)
        .reshape(B * MAXR)
    )
    out = _gather_kernel(x, p_perm)
    return out.reshape(B, 1, 1)


# speedup vs baseline: 1.0411x; 1.0029x over previous
import functools

import jax
import jax.numpy as jnp
from jax import lax
from jax.experimental import pallas as pl
from jax.experimental.pallas import tpu as pltpu
from jax.experimental.pallas import tpu_sc as plsc

MAXR = 1000
B = 16384
NC = 1
NS = 16
NW = NC * NS
BPW = B // NW
CH = 128
NCH = BPW // CH
L = 16
MAGIC = 2.0 ** 23

_mesh = plsc.VectorSubcoreMesh(core_axis_name="c", subcore_axis_name="s", num_cores=1)


@functools.partial(
    pl.kernel,
    mesh=_mesh,
    out_type=jax.ShapeDtypeStruct((NW, NCH, CH), jnp.float32),
    scratch_types=[
        pltpu.VMEM((BPW,), jnp.float32),
        pltpu.VMEM((NCH, CH), jnp.int32),
        pltpu.VMEM((NCH, CH), jnp.float32),
        pltpu.SemaphoreType.DMA,
        pltpu.SemaphoreType.DMA,
        pltpu.SemaphoreType.DMA,
        pltpu.SemaphoreType.DMA,
        pltpu.SemaphoreType.DMA,
        pltpu.SemaphoreType.DMA,
        pltpu.SemaphoreType.DMA,
        pltpu.SemaphoreType.DMA,
        pltpu.SemaphoreType.DMA,
    ],
)
def _gather_kernel(x_hbm, p_hbm, out_hbm, x_v, idx_v, g_v,
                   sg0, sg1, sg2, sg3, sg4, sg5, sg6, sg7, so):
    wid = lax.axis_index("s")
    base = wid * BPW
    sg = (sg0, sg1, sg2, sg3, sg4, sg5, sg6, sg7)
    pltpu.sync_copy(x_hbm.at[pl.ds(base, BPW)], x_v)
    lane = lax.iota(jnp.int32, L)
    gathers = []
    for j in range(NCH):
        for t in range(CH // L):
            off = j * CH + t * L
            v = x_v[pl.ds(off, L)] * float(MAXR - 1)
            r = (v + MAGIC) - MAGIC
            c = r.astype(jnp.int32)
            bconst = (((base + off) >> 7) << 10) + ((base + off) & 127)
            poff = (((c >> 3) << 17) | ((c & 7) << 7)) + (lane + bconst)
            idx_v[j, pl.ds(t * L, L)] = poff
        gathers.append(pltpu.async_copy(p_hbm.at[idx_v.at[j]], g_v.at[j], sg[j]))
    outs = []
    for j in range(NCH):
        gathers[j].wait()
        outs.append(pltpu.async_copy(g_v.at[j], out_hbm.at[wid].at[j], so))
    for o in outs:
        o.wait()


def kernel(x, param):
    p_perm = (
        param.reshape(B, MAXR)
        .transpose(1, 0)
        .reshape(MAXR // 8, 8, B // 128, 128)
        .transpose(0, 2, 1, 3)
        .reshape(B * MAXR)
    )
    out = _gather_kernel(x, p_perm)
    return out.reshape(B, 1, 1)


# speedup vs baseline: 1.0412x; 1.0001x over previous
import functools

import jax
import jax.numpy as jnp
from jax import lax
from jax.experimental import pallas as pl
from jax.experimental.pallas import tpu as pltpu
from jax.experimental.pallas import tpu_sc as plsc

MAXR = 1000
B = 16384
NC = 1
NS = 16
NW = NC * NS
BPW = B // NW
CH = 128
NCH = BPW // CH
L = 16
MAGIC = 2.0 ** 23

_mesh = plsc.VectorSubcoreMesh(core_axis_name="c", subcore_axis_name="s", num_cores=1)


@functools.partial(
    pl.kernel,
    mesh=_mesh,
    out_type=jax.ShapeDtypeStruct((NW, NCH, CH), jnp.float32),
    scratch_types=[
        pltpu.VMEM((BPW,), jnp.float32),
        pltpu.VMEM((NCH, CH), jnp.int32),
        pltpu.VMEM((NCH, CH), jnp.float32),
        pltpu.SemaphoreType.DMA,
        pltpu.SemaphoreType.DMA,
        pltpu.SemaphoreType.DMA,
        pltpu.SemaphoreType.DMA,
        pltpu.SemaphoreType.DMA,
        pltpu.SemaphoreType.DMA,
        pltpu.SemaphoreType.DMA,
        pltpu.SemaphoreType.DMA,
        pltpu.SemaphoreType.DMA,
    ],
)
def _gather_kernel(x_hbm, p_hbm, out_hbm, x_v, idx_v, g_v,
                   sg0, sg1, sg2, sg3, sg4, sg5, sg6, sg7, so):
    wid = lax.axis_index("s")
    base = wid * BPW
    sg = (sg0, sg1, sg2, sg3, sg4, sg5, sg6, sg7)

    @pl.when(lax.axis_index("c") == 0)
    def _body():
        _work(x_hbm, p_hbm, out_hbm, x_v, idx_v, g_v, sg, so, base, wid)


def _work(x_hbm, p_hbm, out_hbm, x_v, idx_v, g_v, sg, so, base, wid):
    pltpu.sync_copy(x_hbm.at[pl.ds(base, BPW)], x_v)
    lane = lax.iota(jnp.int32, L)
    gathers = []
    for j in range(NCH):
        for t in range(CH // L):
            off = j * CH + t * L
            v = x_v[pl.ds(off, L)] * float(MAXR - 1)
            r = (v + MAGIC) - MAGIC
            c = r.astype(jnp.int32)
            bconst = (((base + off) >> 7) << 10) + ((base + off) & 127)
            poff = (((c >> 3) << 17) | ((c & 7) << 7)) + (lane + bconst)
            idx_v[j, pl.ds(t * L, L)] = poff
        gathers.append(pltpu.async_copy(p_hbm.at[idx_v.at[j]], g_v.at[j], sg[j]))
    outs = []
    for j in range(NCH):
        gathers[j].wait()
        outs.append(pltpu.async_copy(g_v.at[j], out_hbm.at[wid].at[j], so))
    for o in outs:
        o.wait()


def kernel(x, param):
    p_perm = (
        param.reshape(B, MAXR)
        .transpose(1, 0)
        .reshape(MAXR // 8, 8, B // 128, 128)
        .transpose(0, 2, 1, 3)
        .reshape(B * MAXR)
    )
    out = _gather_kernel(x, p_perm)
    return out.reshape(B, 1, 1)
